# Initial kernel scaffold; baseline (speedup 1.0000x reference)
import jax
import jax.numpy as jnp
from jax.experimental import pallas as pl


def kernel(atomic_numbers, neighbors):
    raise NotImplementedError("write your pallas kernel here")


# speedup vs baseline: 411.0724x; 411.0724x over previous
import functools

import jax
import jax.numpy as jnp
from jax import lax
from jax.experimental import pallas as pl
from jax.experimental.pallas import tpu as pltpu
from jax.experimental.pallas import tpu_sc as plsc

B, NAT, NNEIGH = 32, 4096, 64
N = NAT * NNEIGH
CHUNK = 16384
NCHUNK = N // CHUNK

_info = plsc.get_sparse_core_info()
NC, NS = _info.num_cores, _info.num_subcores

_mesh = plsc.VectorSubcoreMesh(core_axis_name="c", subcore_axis_name="s")


@functools.partial(
    pl.kernel,
    out_type=jax.ShapeDtypeStruct((B, N), jnp.float32),
    mesh=_mesh,
    scratch_types=[
        pltpu.VMEM((NAT,), jnp.float32),
        pltpu.VMEM((CHUNK,), jnp.int32),
        pltpu.VMEM((CHUNK,), jnp.float32),
    ],
    compiler_params=pltpu.CompilerParams(needs_layout_passes=False),
)
def _sc_gather(tab_hbm, idx_hbm, out_hbm, tab_v, idx_v, out_v):
    wid = lax.axis_index("s") * NC + lax.axis_index("c")
    pltpu.sync_copy(tab_hbm.at[wid], tab_v)
    for c in range(NCHUNK):
        off = c * CHUNK
        pltpu.sync_copy(idx_hbm.at[wid, pl.ds(off, CHUNK)], idx_v)

        @plsc.parallel_loop(0, CHUNK, step=16, unroll=8)
        def _body(i):
            ids = idx_v[pl.ds(i, 16)]
            out_v[pl.ds(i, 16)] = plsc.load_gather(tab_v, [ids])

        pltpu.sync_copy(out_v, out_hbm.at[wid, pl.ds(off, CHUNK)])


def kernel(atomic_numbers, neighbors):
    tab = atomic_numbers.reshape(B, NAT)
    idx = neighbors.reshape(B, N)
    out = _sc_gather(tab, idx)
    return out.reshape(B, NAT, NNEIGH, 1)


# speedup vs baseline: 433.6356x; 1.0549x over previous
import functools

import jax
import jax.numpy as jnp
from jax import lax
from jax.experimental import pallas as pl
from jax.experimental.pallas import tpu as pltpu
from jax.experimental.pallas import tpu_sc as plsc

B, NAT, NNEIGH = 32, 4096, 64
N = NAT * NNEIGH
CHUNK = 16384
NCHUNK = N // CHUNK

_info = plsc.get_sparse_core_info()
NC, NS = _info.num_cores, _info.num_subcores

_mesh = plsc.VectorSubcoreMesh(core_axis_name="c", subcore_axis_name="s")


@functools.partial(
    pl.kernel,
    out_type=jax.ShapeDtypeStruct((B, N), jnp.float32),
    mesh=_mesh,
    scratch_types=[
        pltpu.VMEM((NAT,), jnp.float32),
        pltpu.VMEM((2, CHUNK), jnp.int32),
        pltpu.VMEM((2, CHUNK), jnp.float32),
        pltpu.SemaphoreType.DMA,
        pltpu.SemaphoreType.DMA,
        pltpu.SemaphoreType.DMA,
        pltpu.SemaphoreType.DMA,
    ],
    compiler_params=pltpu.CompilerParams(needs_layout_passes=False),
)
def _sc_gather(tab_hbm, idx_hbm, out_hbm, tab_v, idx_v, out_v,
               in_sem0, in_sem1, out_sem0, out_sem1):
    wid = lax.axis_index("s") * NC + lax.axis_index("c")
    in_sems = (in_sem0, in_sem1)
    out_sems = (out_sem0, out_sem1)
    pltpu.sync_copy(tab_hbm.at[wid], tab_v)

    in_copies = [None] * NCHUNK
    out_copies = [None] * NCHUNK
    in_copies[0] = pltpu.async_copy(
        idx_hbm.at[wid, pl.ds(0, CHUNK)], idx_v.at[0], in_sems[0])
    for c in range(NCHUNK):
        buf = c % 2
        if c + 1 < NCHUNK:
            nbuf = (c + 1) % 2
            in_copies[c + 1] = pltpu.async_copy(
                idx_hbm.at[wid, pl.ds((c + 1) * CHUNK, CHUNK)],
                idx_v.at[nbuf], in_sems[nbuf])
        in_copies[c].wait()
        if c >= 2:
            out_copies[c - 2].wait()

        @plsc.parallel_loop(0, CHUNK, step=16, unroll=8)
        def _body(i):
            ids = idx_v[buf, pl.ds(i, 16)]
            out_v[buf, pl.ds(i, 16)] = plsc.load_gather(tab_v, [ids])

        out_copies[c] = pltpu.async_copy(
            out_v.at[buf], out_hbm.at[wid, pl.ds(c * CHUNK, CHUNK)],
            out_sems[buf])
    out_copies[NCHUNK - 2].wait()
    out_copies[NCHUNK - 1].wait()


def kernel(atomic_numbers, neighbors):
    tab = atomic_numbers.reshape(B, NAT)
    idx = neighbors.reshape(B, N)
    out = _sc_gather(tab, idx)
    return out.reshape(B, NAT, NNEIGH, 1)


# speedup vs baseline: 683.6786x; 1.5766x over previous
import functools

import jax
import jax.numpy as jnp
from jax import lax
from jax.experimental import pallas as pl
from jax.experimental.pallas import tpu as pltpu
from jax.experimental.pallas import tpu_sc as plsc

B, NAT, NNEIGH = 32, 4096, 64
ROWS = 128
CHUNK = ROWS * NNEIGH
NCHUNK = NAT // ROWS

_info = plsc.get_sparse_core_info()
NC, NS = _info.num_cores, _info.num_subcores

_mesh = plsc.VectorSubcoreMesh(core_axis_name="c", subcore_axis_name="s")


@functools.partial(
    pl.kernel,
    out_type=jax.ShapeDtypeStruct((B, NAT, NNEIGH), jnp.float32),
    mesh=_mesh,
    scratch_types=[
        pltpu.VMEM((NAT,), jnp.float32),
        pltpu.VMEM((2, ROWS, NNEIGH), jnp.int32),
        pltpu.VMEM((2, ROWS, NNEIGH), jnp.float32),
        pltpu.SemaphoreType.DMA,
        pltpu.SemaphoreType.DMA,
        pltpu.SemaphoreType.DMA,
        pltpu.SemaphoreType.DMA,
    ],
    compiler_params=pltpu.CompilerParams(needs_layout_passes=False),
)
def _sc_gather(tab_hbm, idx_hbm, out_hbm, tab_v, idx_v, out_v,
               in_sem0, in_sem1, out_sem0, out_sem1):
    wid = lax.axis_index("s") * NC + lax.axis_index("c")
    in_sems = (in_sem0, in_sem1)
    out_sems = (out_sem0, out_sem1)
    pltpu.sync_copy(tab_hbm.at[wid], tab_v)

    in_copies = [None] * NCHUNK
    out_copies = [None] * NCHUNK
    in_copies[0] = pltpu.async_copy(
        idx_hbm.at[wid, pl.ds(0, ROWS), :], idx_v.at[0], in_sems[0])
    for c in range(NCHUNK):
        buf = c % 2
        if c + 1 < NCHUNK:
            nbuf = (c + 1) % 2
            in_copies[c + 1] = pltpu.async_copy(
                idx_hbm.at[wid, pl.ds((c + 1) * ROWS, ROWS), :],
                idx_v.at[nbuf], in_sems[nbuf])
        in_copies[c].wait()
        if c >= 2:
            out_copies[c - 2].wait()

        @plsc.parallel_loop(0, ROWS, step=1, unroll=8)
        def _body(r):
            for j in range(0, NNEIGH, 16):
                ids = idx_v[buf, r, pl.ds(j, 16)]
                out_v[buf, r, pl.ds(j, 16)] = plsc.load_gather(tab_v, [ids])

        out_copies[c] = pltpu.async_copy(
            out_v.at[buf],
            out_hbm.at[wid, pl.ds(c * ROWS, ROWS), :],
            out_sems[buf])
    out_copies[NCHUNK - 2].wait()
    out_copies[NCHUNK - 1].wait()


def kernel(atomic_numbers, neighbors):
    tab = atomic_numbers.reshape(B, NAT)
    return _sc_gather(tab, neighbors)[..., None]
